# Initial kernel scaffold; baseline (speedup 1.0000x reference)
import jax
import jax.numpy as jnp
from jax.experimental import pallas as pl


def kernel(x):
    raise NotImplementedError("write your pallas kernel here")


# speedup vs baseline: 50.1907x; 50.1907x over previous
import functools

import jax
import jax.numpy as jnp
from jax import lax
from jax.experimental import pallas as pl
from jax.experimental.pallas import tpu as pltpu
from jax.experimental.pallas import tpu_sc as plsc

LANES = 16
BINS = 256
CHUNK = 32768


def _sc_hist_body(x_hbm, out_hbm, buf0, buf1, hist, merged, sem0, sem1):
    c = lax.axis_index("c")
    s = lax.axis_index("s")
    row = s
    half = c
    n = x_hbm.shape[1]
    half_n = n // 2
    nchunk = half_n // CHUNK
    base = half * half_n

    zeros = jnp.zeros((LANES,), jnp.float32)

    def zbody(i, _):
        hist[pl.ds(i * LANES, LANES)] = zeros
        return 0

    lax.fori_loop(0, (LANES * BINS) // LANES, zbody, 0)

    lane_off = lax.iota(jnp.int32, LANES) * BINS
    ones = jnp.ones((LANES,), jnp.float32)

    def chunk_loop(buf):
        def body(i, _):
            v = buf[pl.ds(i * LANES, LANES)]
            idx = jnp.clip(v.astype(jnp.int32), 0, BINS - 1) + lane_off
            plsc.addupdate_scatter(hist, [idx], ones)
            return 0

        lax.fori_loop(0, CHUNK // LANES, body, 0)

    cp = pltpu.async_copy(x_hbm.at[row, pl.ds(base, CHUNK)], buf0, sem0)
    for g in range(nchunk):
        buf, nbuf, nsem = (buf0, buf1, sem1) if g % 2 == 0 else (buf1, buf0, sem0)
        ncp = None
        if g + 1 < nchunk:
            ncp = pltpu.async_copy(
                x_hbm.at[row, pl.ds(base + (g + 1) * CHUNK, CHUNK)], nbuf, nsem
            )
        cp.wait()
        chunk_loop(buf)
        cp = ncp

    for g in range(BINS // LANES):
        acc = hist[pl.ds(g * LANES, LANES)]
        for l in range(1, LANES):
            acc = acc + hist[pl.ds(l * BINS + g * LANES, LANES)]
        merged[pl.ds(g * LANES, LANES)] = acc

    pltpu.sync_copy(merged, out_hbm.at[row, half])


def _tc_bcast_body(p_ref, o_ref):
    h = p_ref[0, 0, :] + p_ref[0, 1, :]
    o_ref[0] = jnp.broadcast_to(h[:, None], (BINS, BINS))


@jax.jit
def kernel(x):
    b, n = x.shape
    sc_hist = pl.kernel(
        _sc_hist_body,
        out_type=jax.ShapeDtypeStruct((b, 2, BINS), jnp.float32),
        mesh=plsc.VectorSubcoreMesh(core_axis_name="c", subcore_axis_name="s"),
        scratch_types=[
            pltpu.VMEM((CHUNK,), jnp.float32),
            pltpu.VMEM((CHUNK,), jnp.float32),
            pltpu.VMEM((LANES * BINS,), jnp.float32),
            pltpu.VMEM((BINS,), jnp.float32),
            pltpu.SemaphoreType.DMA,
            pltpu.SemaphoreType.DMA,
        ],
        compiler_params=pltpu.CompilerParams(needs_layout_passes=False),
    )
    partials = sc_hist(x)
    out = pl.pallas_call(
        _tc_bcast_body,
        grid=(b,),
        in_specs=[pl.BlockSpec((1, 2, BINS), lambda i: (i, 0, 0))],
        out_specs=pl.BlockSpec((1, BINS, BINS), lambda i: (i, 0, 0)),
        out_shape=jax.ShapeDtypeStruct((b, BINS, BINS), jnp.float32),
    )(partials)
    return out


# speedup vs baseline: 180.7500x; 3.6013x over previous
import functools

import jax
import jax.numpy as jnp
from jax import lax
from jax.experimental import pallas as pl
from jax.experimental.pallas import tpu as pltpu
from jax.experimental.pallas import tpu_sc as plsc

LANES = 16
BINS = 256
CHUNK = 32768


def _sc_hist_body(x_hbm, out_hbm, buf0, buf1, hist, merged, sem0, sem1):
    c = lax.axis_index("c")
    s = lax.axis_index("s")
    row = s
    half = c
    n = x_hbm.shape[1]
    half_n = n // 2
    nchunk = half_n // CHUNK
    base = half * half_n

    zeros = jnp.zeros((LANES,), jnp.float32)

    @plsc.parallel_loop(0, (LANES * BINS) // LANES, unroll=8)
    def _(i):
        hist[pl.ds(i * LANES, LANES)] = zeros

    lane_off = lax.iota(jnp.int32, LANES) * BINS
    ones = jnp.ones((LANES,), jnp.float32)

    def chunk_loop(buf):
        @plsc.parallel_loop(0, CHUNK // LANES, unroll=8)
        def _(i):
            v = buf[pl.ds(i * LANES, LANES)]
            idx = jnp.clip(v.astype(jnp.int32), 0, BINS - 1) + lane_off
            plsc.addupdate_scatter(hist, [idx], ones)

    cp = pltpu.async_copy(x_hbm.at[row, pl.ds(base, CHUNK)], buf0, sem0)
    for g in range(nchunk):
        buf, nbuf, nsem = (buf0, buf1, sem1) if g % 2 == 0 else (buf1, buf0, sem0)
        ncp = None
        if g + 1 < nchunk:
            ncp = pltpu.async_copy(
                x_hbm.at[row, pl.ds(base + (g + 1) * CHUNK, CHUNK)], nbuf, nsem
            )
        cp.wait()
        chunk_loop(buf)
        cp = ncp

    for g in range(BINS // LANES):
        acc = hist[pl.ds(g * LANES, LANES)]
        for l in range(1, LANES):
            acc = acc + hist[pl.ds(l * BINS + g * LANES, LANES)]
        merged[pl.ds(g * LANES, LANES)] = acc

    pltpu.sync_copy(merged, out_hbm.at[row, half])


def _tc_bcast_body(p_ref, o_ref):
    h = p_ref[0, 0, :] + p_ref[0, 1, :]
    o_ref[0] = jnp.broadcast_to(h[:, None], (BINS, BINS))


@jax.jit
def kernel(x):
    b, n = x.shape
    sc_hist = pl.kernel(
        _sc_hist_body,
        out_type=jax.ShapeDtypeStruct((b, 2, BINS), jnp.float32),
        mesh=plsc.VectorSubcoreMesh(core_axis_name="c", subcore_axis_name="s"),
        scratch_types=[
            pltpu.VMEM((CHUNK,), jnp.float32),
            pltpu.VMEM((CHUNK,), jnp.float32),
            pltpu.VMEM((LANES * BINS,), jnp.float32),
            pltpu.VMEM((BINS,), jnp.float32),
            pltpu.SemaphoreType.DMA,
            pltpu.SemaphoreType.DMA,
        ],
        compiler_params=pltpu.CompilerParams(needs_layout_passes=False),
    )
    partials = sc_hist(x)
    out = pl.pallas_call(
        _tc_bcast_body,
        grid=(b,),
        in_specs=[pl.BlockSpec((1, 2, BINS), lambda i: (i, 0, 0))],
        out_specs=pl.BlockSpec((1, BINS, BINS), lambda i: (i, 0, 0)),
        out_shape=jax.ShapeDtypeStruct((b, BINS, BINS), jnp.float32),
    )(partials)
    return out


# speedup vs baseline: 186.1005x; 1.0296x over previous
import functools

import jax
import jax.numpy as jnp
from jax import lax
from jax.experimental import pallas as pl
from jax.experimental.pallas import tpu as pltpu
from jax.experimental.pallas import tpu_sc as plsc

LANES = 16
BINS = 256
CHUNK = 32768


def _sc_hist_body(x_hbm, out_hbm, buf0, buf1, hist, merged, sem0, sem1):
    c = lax.axis_index("c")
    s = lax.axis_index("s")
    row = s
    half = c
    n = x_hbm.shape[1]
    half_n = n // 2
    nchunk = half_n // CHUNK
    base = half * half_n

    zeros = jnp.zeros((LANES,), jnp.float32)

    @plsc.parallel_loop(0, (LANES * BINS) // LANES, unroll=8)
    def _(i):
        hist[pl.ds(i * LANES, LANES)] = zeros

    lane_off = lax.iota(jnp.int32, LANES) * BINS
    ones = jnp.ones((LANES,), jnp.float32)

    def chunk_loop(buf):
        @plsc.parallel_loop(0, CHUNK // LANES, unroll=16)
        def _(i):
            v = buf[pl.ds(i * LANES, LANES)]
            idx = v.astype(jnp.int32) + lane_off
            plsc.addupdate_scatter(hist, [idx], ones)

    cp = pltpu.async_copy(x_hbm.at[row, pl.ds(base, CHUNK)], buf0, sem0)
    for g in range(nchunk):
        buf, nbuf, nsem = (buf0, buf1, sem1) if g % 2 == 0 else (buf1, buf0, sem0)
        ncp = None
        if g + 1 < nchunk:
            ncp = pltpu.async_copy(
                x_hbm.at[row, pl.ds(base + (g + 1) * CHUNK, CHUNK)], nbuf, nsem
            )
        cp.wait()
        chunk_loop(buf)
        cp = ncp

    for g in range(BINS // LANES):
        acc = hist[pl.ds(g * LANES, LANES)]
        for l in range(1, LANES):
            acc = acc + hist[pl.ds(l * BINS + g * LANES, LANES)]
        merged[pl.ds(g * LANES, LANES)] = acc

    pltpu.sync_copy(merged, out_hbm.at[row, half])


def _tc_bcast_body(p_ref, o_ref):
    h = p_ref[0, 0, :] + p_ref[0, 1, :]
    o_ref[0] = jnp.broadcast_to(h[:, None], (BINS, BINS))


@jax.jit
def kernel(x):
    b, n = x.shape
    sc_hist = pl.kernel(
        _sc_hist_body,
        out_type=jax.ShapeDtypeStruct((b, 2, BINS), jnp.float32),
        mesh=plsc.VectorSubcoreMesh(core_axis_name="c", subcore_axis_name="s"),
        scratch_types=[
            pltpu.VMEM((CHUNK,), jnp.float32),
            pltpu.VMEM((CHUNK,), jnp.float32),
            pltpu.VMEM((LANES * BINS,), jnp.float32),
            pltpu.VMEM((BINS,), jnp.float32),
            pltpu.SemaphoreType.DMA,
            pltpu.SemaphoreType.DMA,
        ],
        compiler_params=pltpu.CompilerParams(needs_layout_passes=False),
    )
    partials = sc_hist(x)
    out = pl.pallas_call(
        _tc_bcast_body,
        grid=(b,),
        in_specs=[pl.BlockSpec((1, 2, BINS), lambda i: (i, 0, 0))],
        out_specs=pl.BlockSpec((1, BINS, BINS), lambda i: (i, 0, 0)),
        out_shape=jax.ShapeDtypeStruct((b, BINS, BINS), jnp.float32),
    )(partials)
    return out


# speedup vs baseline: 221.0458x; 1.1878x over previous
import functools

import jax
import jax.numpy as jnp
from jax import lax
from jax.experimental import pallas as pl
from jax.experimental.pallas import tpu as pltpu
from jax.experimental.pallas import tpu_sc as plsc

LANES = 16
BINS = 256
CHUNK = 32768


def _sc_hist_body(x_hbm, out_hbm, buf0, buf1, hist, sem0, sem1):
    c = lax.axis_index("c")
    s = lax.axis_index("s")
    row = s
    half = c
    n = x_hbm.shape[1]
    half_n = n // 2
    nchunk = half_n // CHUNK
    base = half * half_n

    zeros = jnp.zeros((LANES,), jnp.float32)

    @plsc.parallel_loop(0, BINS, unroll=8)
    def _(i):
        hist[i, :] = zeros

    lane_off = lax.iota(jnp.int32, LANES)
    ones = jnp.ones((LANES,), jnp.float32)

    def chunk_loop(buf):
        @plsc.parallel_loop(0, CHUNK // LANES, unroll=16)
        def _(i):
            v = buf[pl.ds(i * LANES, LANES)]
            plsc.addupdate_scatter(hist, [v.astype(jnp.int32), lane_off], ones)

    cp = pltpu.async_copy(x_hbm.at[row, pl.ds(base, CHUNK)], buf0, sem0)
    for g in range(nchunk):
        buf, nbuf, nsem = (buf0, buf1, sem1) if g % 2 == 0 else (buf1, buf0, sem0)
        ncp = None
        if g + 1 < nchunk:
            ncp = pltpu.async_copy(
                x_hbm.at[row, pl.ds(base + (g + 1) * CHUNK, CHUNK)], nbuf, nsem
            )
        cp.wait()
        chunk_loop(buf)
        cp = ncp

    pltpu.sync_copy(hist, out_hbm.at[row, half])


def _tc_bcast_body(p_ref, o_ref):
    h = jnp.sum(p_ref[0], axis=(0, 2))
    o_ref[0] = jnp.broadcast_to(h[:, None], (BINS, BINS))


@jax.jit
def kernel(x):
    b, n = x.shape
    sc_hist = pl.kernel(
        _sc_hist_body,
        out_type=jax.ShapeDtypeStruct((b, 2, BINS, LANES), jnp.float32),
        mesh=plsc.VectorSubcoreMesh(core_axis_name="c", subcore_axis_name="s"),
        scratch_types=[
            pltpu.VMEM((CHUNK,), jnp.float32),
            pltpu.VMEM((CHUNK,), jnp.float32),
            pltpu.VMEM((BINS, LANES), jnp.float32),
            pltpu.SemaphoreType.DMA,
            pltpu.SemaphoreType.DMA,
        ],
        compiler_params=pltpu.CompilerParams(needs_layout_passes=False),
    )
    partials = sc_hist(x)
    out = pl.pallas_call(
        _tc_bcast_body,
        grid=(b,),
        in_specs=[pl.BlockSpec((1, 2, BINS, LANES), lambda i: (i, 0, 0, 0))],
        out_specs=pl.BlockSpec((1, BINS, BINS), lambda i: (i, 0, 0)),
        out_shape=jax.ShapeDtypeStruct((b, BINS, BINS), jnp.float32),
    )(partials)
    return out


# speedup vs baseline: 235.0562x; 1.0634x over previous
import jax
import jax.numpy as jnp
from jax import lax
from jax.experimental import pallas as pl
from jax.experimental.pallas import tpu as pltpu
from jax.experimental.pallas import tpu_sc as plsc

LANES = 16
BINS = 256
CHUNK = 16384


def _sc_body(x_hbm, out_hbm, buf0, buf1, hist, merged, pbuf, outbuf, shared,
             sem0, sem1):
    c = lax.axis_index("c")
    s = lax.axis_index("s")
    row = 8 * c + (s // 2)
    half = s % 2
    n = x_hbm.shape[1]
    half_n = n // 2
    nchunk = half_n // CHUNK
    base = half * half_n

    zeros = jnp.zeros((LANES,), jnp.float32)

    @plsc.parallel_loop(0, BINS, unroll=8)
    def _(i):
        hist[i, :] = zeros

    @plsc.parallel_loop(0, BINS // LANES, unroll=4)
    def _(i):
        merged[pl.ds(i * LANES, LANES)] = zeros

    lane_off = lax.iota(jnp.int32, LANES)
    ones = jnp.ones((LANES,), jnp.float32)

    def chunk_loop(buf):
        @plsc.parallel_loop(0, CHUNK // LANES, unroll=16)
        def _(i):
            v = buf[pl.ds(i * LANES, LANES)]
            plsc.addupdate_scatter(hist, [v.astype(jnp.int32), lane_off], ones)

    cp = pltpu.async_copy(x_hbm.at[row, pl.ds(base, CHUNK)], buf0, sem0)
    for g in range(nchunk):
        buf, nbuf, nsem = (buf0, buf1, sem1) if g % 2 == 0 else (buf1, buf0, sem0)
        ncp = None
        if g + 1 < nchunk:
            ncp = pltpu.async_copy(
                x_hbm.at[row, pl.ds(base + (g + 1) * CHUNK, CHUNK)], nbuf, nsem
            )
        cp.wait()
        chunk_loop(buf)
        cp = ncp

    def fold_loop(_):
        @plsc.parallel_loop(0, BINS, unroll=4)
        def _(b):
            idx = lax.broadcast(b, (LANES,))
            plsc.addupdate_scatter(merged, [idx], hist[b, :])

    fold_loop(None)

    pltpu.sync_copy(merged, shared.at[s])
    plsc.subcore_barrier()
    pltpu.sync_copy(shared.at[s ^ 1], pbuf)

    @plsc.parallel_loop(0, BINS // LANES, unroll=4)
    def _(i):
        sl = pl.ds(i * LANES, LANES)
        merged[sl] = merged[sl] + pbuf[sl]

    gb0 = half * (BINS // 2)
    for piece in range(2):
        pb0 = gb0 + piece * (BINS // 4)

        @plsc.parallel_loop(0, BINS // 4, unroll=2)
        def _(b):
            idx = lax.broadcast(pb0 + b, (LANES,))
            tot = plsc.load_gather(merged, [idx])
            for k in range(BINS // LANES):
                outbuf[b, pl.ds(k * LANES, LANES)] = tot

        pltpu.sync_copy(outbuf, out_hbm.at[row, pl.ds(pb0, BINS // 4)])


@jax.jit
def kernel(x):
    b, n = x.shape
    sc_hist = pl.kernel(
        _sc_body,
        out_type=jax.ShapeDtypeStruct((b, BINS, BINS), jnp.float32),
        mesh=plsc.VectorSubcoreMesh(core_axis_name="c", subcore_axis_name="s"),
        scratch_types=[
            pltpu.VMEM((CHUNK,), jnp.float32),
            pltpu.VMEM((CHUNK,), jnp.float32),
            pltpu.VMEM((BINS, LANES), jnp.float32),
            pltpu.VMEM((BINS,), jnp.float32),
            pltpu.VMEM((BINS,), jnp.float32),
            pltpu.VMEM((BINS // 4, BINS), jnp.float32),
            pltpu.VMEM_SHARED((LANES, BINS), jnp.float32),
            pltpu.SemaphoreType.DMA,
            pltpu.SemaphoreType.DMA,
        ],
        compiler_params=pltpu.CompilerParams(needs_layout_passes=False),
    )
    return sc_hist(x)
